# Initial kernel scaffold; baseline (speedup 1.0000x reference)
import jax
import jax.numpy as jnp
from jax.experimental import pallas as pl


def kernel(x, edge_index, params, neg_ratio):
    raise NotImplementedError("write your pallas kernel here")


# speedup vs baseline: 4.4925x; 4.4925x over previous
import functools

import jax
import jax.numpy as jnp
from jax import lax
from jax.experimental import pallas as pl
from jax.experimental.pallas import tpu as pltpu
from jax.experimental.pallas import tpu_sc as plsc

S = 64
IN = 128
Z = 32
G = 16
RE = 6
RD = 2
N = 10000
E = 320000

ROW_BLK = 1000
N_ROW_BLKS = N // ROW_BLK

NC = 2
NS = 16
NW = NC * NS

EPW = E // NW
CH = 80
NCH = EPW // CH
NP = 10240
NPS = NP // NS

EG = 4 * E
GPW = EG // NW
GCH = 80
GNCH = GPW // GCH

EDGE_BLK = 4000
N_EDGE_BLKS = (2 * E) // EDGE_BLK
N_POS_BLKS = E // EDGE_BLK


def _dot(a, b):
    return jnp.dot(a, b, preferred_element_type=jnp.float32)


def _full_spec(shape):
    return pl.BlockSpec(shape, lambda i: (0,) * len(shape))


def _row_spec(last):
    return pl.BlockSpec((ROW_BLK, last), lambda i: (i, 0))


def _in_proj_kernel(x_ref, w_ref, b_ref, mw_ref, mb_ref, h_ref, m_ref):
    h = jnp.maximum(_dot(x_ref[...], w_ref[...]) + b_ref[...], 0.0)
    h_ref[...] = h
    m_ref[...] = jnp.maximum(_dot(h, mw_ref[...]) + mb_ref[...], 0.0)


def _in_proj(x, wt, brow, mwt, mbrow):
    return pl.pallas_call(
        _in_proj_kernel,
        grid=(N_ROW_BLKS,),
        in_specs=[
            _row_spec(IN),
            _full_spec((IN, S)),
            _full_spec((1, S)),
            _full_spec((S, S)),
            _full_spec((1, S)),
        ],
        out_specs=[_row_spec(S), _row_spec(S)],
        out_shape=[
            jax.ShapeDtypeStruct((N, S), jnp.float32),
            jax.ShapeDtypeStruct((N, S), jnp.float32),
        ],
    )(x, wt, brow, mwt, mbrow)


def _upd_kernel(fuse_next, a0_ref, a1_ref, h_ref, uw_ref, ub_ref,
                wr_ref, wz_ref, wn_ref, ur_ref, uz_ref, un_ref,
                brz_ref, bzz_ref, bin_ref, bhn_ref, nw_ref, nb_ref,
                *out_refs):
    agg = a0_ref[...] + a1_ref[...]
    msg = jnp.maximum(_dot(agg, uw_ref[...]) + ub_ref[...], 0.0)
    h = h_ref[...]
    r = jax.nn.sigmoid(_dot(msg, wr_ref[...]) + _dot(h, ur_ref[...]) + brz_ref[...])
    z = jax.nn.sigmoid(_dot(msg, wz_ref[...]) + _dot(h, uz_ref[...]) + bzz_ref[...])
    n = jnp.tanh(_dot(msg, wn_ref[...]) + bin_ref[...]
                 + r * (_dot(h, un_ref[...]) + bhn_ref[...]))
    hn = (1.0 - z) * n + z * h
    out_refs[0][...] = hn
    if fuse_next:
        out_refs[1][...] = jnp.maximum(_dot(hn, nw_ref[...]) + nb_ref[...], 0.0)


def _upd(a0, a1, h, uwt, ubrow, gw, nwt, nbrow):
    fuse_next = nwt is not None
    if not fuse_next:
        nwt = jnp.zeros((S, S), jnp.float32)
        nbrow = jnp.zeros((1, S), jnp.float32)
    n_out = 2 if fuse_next else 1
    outs = pl.pallas_call(
        functools.partial(_upd_kernel, fuse_next),
        grid=(N_ROW_BLKS,),
        in_specs=[
            _row_spec(S),
            _row_spec(S),
            _row_spec(S),
            _full_spec((S, S)), _full_spec((1, S)),
            _full_spec((S, S)), _full_spec((S, S)), _full_spec((S, S)),
            _full_spec((S, S)), _full_spec((S, S)), _full_spec((S, S)),
            _full_spec((1, S)), _full_spec((1, S)), _full_spec((1, S)),
            _full_spec((1, S)),
            _full_spec((S, S)), _full_spec((1, S)),
        ],
        out_specs=[_row_spec(S)] * n_out,
        out_shape=[jax.ShapeDtypeStruct((N, S), jnp.float32)] * n_out,
    )(a0, a1, h, uwt, ubrow, *gw, nwt, nbrow)
    if fuse_next:
        return outs[0], outs[1]
    return outs[0], None


def _mu_kernel(h_ref, mw_ref, mb_ref, lw_ref, lb_ref,
               mu_ref, lv_ref, sum_ref, kl_ref):
    h = h_ref[...]
    mu = _dot(h, mw_ref[...]) + mb_ref[...]
    lv = _dot(h, lw_ref[...]) + lb_ref[...]
    mu_ref[...] = mu
    lv_ref[...] = lv

    @pl.when(pl.program_id(0) == 0)
    def _():
        sum_ref[...] = jnp.zeros_like(sum_ref)
        kl_ref[...] = jnp.zeros_like(kl_ref)

    sum_ref[...] += jnp.sum(h, axis=0, keepdims=True)
    kl_ref[...] += jnp.sum(1.0 + lv - mu * mu - jnp.exp(lv)).reshape(1, 1)


def _mu_head(h, mwt, mbrow, lwt, lbrow):
    return pl.pallas_call(
        _mu_kernel,
        grid=(N_ROW_BLKS,),
        in_specs=[
            _row_spec(S),
            _full_spec((S, Z)), _full_spec((1, Z)),
            _full_spec((S, Z)), _full_spec((1, Z)),
        ],
        out_specs=[
            _row_spec(Z), _row_spec(Z),
            _full_spec((1, S)), _full_spec((1, 1)),
        ],
        out_shape=[
            jax.ShapeDtypeStruct((N, Z), jnp.float32),
            jax.ShapeDtypeStruct((N, Z), jnp.float32),
            jax.ShapeDtypeStruct((1, S), jnp.float32),
            jax.ShapeDtypeStruct((1, 1), jnp.float32),
        ],
    )(h, mwt, mbrow, lwt, lbrow)


def _z_kernel(mu_ref, lv_ref, eps_ref, pw_ref, be_ref, mw_ref, mb_ref,
              h_ref, m_ref):
    zn = mu_ref[...] + eps_ref[...] * jnp.exp(0.5 * lv_ref[...])
    h = jnp.maximum(_dot(zn, pw_ref[...]) + be_ref[...], 0.0)
    h_ref[...] = h
    m_ref[...] = jnp.maximum(_dot(h, mw_ref[...]) + mb_ref[...], 0.0)


def _z_proj(mu_n, lv_n, eps_n, pwt_n, beff, mwt, mbrow):
    return pl.pallas_call(
        _z_kernel,
        grid=(N_ROW_BLKS,),
        in_specs=[
            _row_spec(Z), _row_spec(Z), _row_spec(Z),
            _full_spec((Z, S)), _full_spec((1, S)),
            _full_spec((S, S)), _full_spec((1, S)),
        ],
        out_specs=[_row_spec(S), _row_spec(S)],
        out_shape=[
            jax.ShapeDtypeStruct((N, S), jnp.float32),
            jax.ShapeDtypeStruct((N, S), jnp.float32),
        ],
    )(mu_n, lv_n, eps_n, pwt_n, beff, mwt, mbrow)


def _edge_kernel(hu_ref, hv_ref, wa_ref, wb_ref, wc_ref, wd_ref,
                 b1_ref, w2_ref, b2_ref, pos_ref, neg_ref):
    hu = hu_ref[...]
    hv = hv_ref[...]
    t = (_dot(hu, wa_ref[...]) + _dot(hv, wb_ref[...])
         + _dot(jnp.abs(hu - hv), wc_ref[...]) + _dot(hu * hv, wd_ref[...])
         + b1_ref[...])
    t = jnp.maximum(t, 0.0)
    logits = _dot(t, w2_ref[...]) + b2_ref[...]

    @pl.when(pl.program_id(0) == 0)
    def _():
        pos_ref[...] = jnp.zeros_like(pos_ref)
        neg_ref[...] = jnp.zeros_like(neg_ref)

    is_pos = pl.program_id(0) < N_POS_BLKS
    pos_term = jnp.sum(jax.nn.log_sigmoid(logits)).reshape(1, 1)
    neg_term = jnp.sum(jax.nn.log_sigmoid(-logits)).reshape(1, 1)
    pos_ref[...] += jnp.where(is_pos, pos_term, jnp.zeros((1, 1), jnp.float32))
    neg_ref[...] += jnp.where(is_pos, jnp.zeros((1, 1), jnp.float32), neg_term)


def _edge_score(gv, wa, wb, wc, wd, b1row, w2col, b2s):
    return pl.pallas_call(
        _edge_kernel,
        grid=(N_EDGE_BLKS,),
        in_specs=[
            pl.BlockSpec((EDGE_BLK, S), lambda i: (i, 0)),
            pl.BlockSpec((EDGE_BLK, S), lambda i: (i + N_EDGE_BLKS, 0)),
            _full_spec((S, S)), _full_spec((S, S)),
            _full_spec((S, S)), _full_spec((S, S)),
            _full_spec((1, S)), _full_spec((S, 1)), _full_spec((1, 1)),
        ],
        out_specs=[_full_spec((1, 1)), _full_spec((1, 1))],
        out_shape=[
            jax.ShapeDtypeStruct((1, 1), jnp.float32),
            jax.ShapeDtypeStruct((1, 1), jnp.float32),
        ],
    )(gv, gv, wa, wb, wc, wd, b1row, w2col, b2s)


@functools.lru_cache(maxsize=None)
def _build_agg_sc():
    mesh = plsc.VectorSubcoreMesh(core_axis_name="c", subcore_axis_name="s")

    @functools.partial(
        pl.kernel,
        mesh=mesh,
        compiler_params=pltpu.CompilerParams(use_tc_tiling_on_sc=False),
        out_type=jax.ShapeDtypeStruct((NC * NP, S), jnp.float32),
        scratch_types=[
            pltpu.VMEM((NCH, CH), jnp.int32),
            pltpu.VMEM((NCH, CH), jnp.int32),
            pltpu.VMEM((CH, S), jnp.float32),
            pltpu.VMEM_SHARED((NP, S), jnp.float32),
            pltpu.SemaphoreType.DMA,
        ],
    )
    def agg_sc(m_hbm, src_hbm, dst_hbm, zeros_hbm, out_hbm,
               idx_s, idx_d, rows, agg_sh, gsem):
        cid = lax.axis_index("c")
        sid = lax.axis_index("s")
        wid = sid * NC + cid
        pltpu.sync_copy(zeros_hbm, agg_sh.at[pl.ds(sid * NPS, NPS)])
        pltpu.sync_copy(src_hbm.at[wid], idx_s)
        pltpu.sync_copy(dst_hbm.at[wid], idx_d)
        plsc.subcore_barrier()

        def body(j, carry):
            pltpu.async_copy(m_hbm.at[idx_s.at[j]], rows, gsem).wait()
            pltpu.sync_copy(rows, agg_sh.at[idx_d.at[j]], add=True)
            return carry

        lax.fori_loop(0, NCH, body, 0)
        plsc.subcore_barrier()
        pltpu.sync_copy(agg_sh.at[pl.ds(sid * NPS, NPS)],
                        out_hbm.at[pl.ds(cid * NP + sid * NPS, NPS)])

    return agg_sc


@functools.lru_cache(maxsize=None)
def _build_gather_sc():
    mesh = plsc.VectorSubcoreMesh(core_axis_name="c", subcore_axis_name="s")

    @functools.partial(
        pl.kernel,
        mesh=mesh,
        compiler_params=pltpu.CompilerParams(use_tc_tiling_on_sc=False),
        out_type=jax.ShapeDtypeStruct((EG, S), jnp.float32),
        scratch_types=[
            pltpu.VMEM((GPW,), jnp.int32),
            pltpu.VMEM((GCH, S), jnp.float32),
            pltpu.SemaphoreType.DMA,
        ],
    )
    def gather_sc(hd_hbm, idx_hbm, out_hbm, idx_v, rows, gsem):
        cid = lax.axis_index("c")
        sid = lax.axis_index("s")
        wid = sid * NC + cid
        pltpu.sync_copy(idx_hbm.at[pl.ds(wid * GPW, GPW)], idx_v)

        def body(j, carry):
            pltpu.async_copy(
                hd_hbm.at[idx_v.at[pl.ds(j * GCH, GCH)]], rows, gsem).wait()
            pltpu.sync_copy(rows, out_hbm.at[pl.ds(wid * GPW + j * GCH, GCH)])
            return carry

        lax.fori_loop(0, GNCH, body, 0)

    return gather_sc


def _row(v):
    return v[None, :]


def kernel(x, edge_index, params, neg_ratio):
    p = params
    f32 = jnp.float32

    eps_n = jax.random.normal(jax.random.key(11), (N, Z), dtype=f32)
    eps_g = jax.random.normal(jax.random.key(12), (G,), dtype=f32)
    neg = jax.random.randint(jax.random.key(13), (2, E), 0, N,
                             dtype=edge_index.dtype)
    neg = neg % (neg_ratio * N)

    src = edge_index[0]
    dst = edge_index[1]
    src3d = src.reshape(NW, NCH, CH)
    dst3d = dst.reshape(NW, NCH, CH)
    zeros_rows = jnp.zeros((NPS, S), f32)

    def grup(prefix):
        wih = p[prefix + '_Wih']; whh = p[prefix + '_Whh']
        bih = p[prefix + '_bih']; bhh = p[prefix + '_bhh']
        return (
            wih[0:S].T, wih[S:2 * S].T, wih[2 * S:].T,
            whh[0:S].T, whh[S:2 * S].T, whh[2 * S:].T,
            _row(bih[0:S] + bhh[0:S]), _row(bih[S:2 * S] + bhh[S:2 * S]),
            _row(bih[2 * S:]), _row(bhh[2 * S:]),
        )

    enc_g = grup('enc_gru')
    dec_g = grup('dec_gru')

    agg_sc = _build_agg_sc()
    gather_sc = _build_gather_sc()

    h, m = _in_proj(x, p['enc_in_W'].T, _row(p['enc_in_b']),
                    p['enc_msg_W'][0].T, _row(p['enc_msg_b'][0]))
    for r in range(RE):
        aggp = agg_sc(m, src3d, dst3d, zeros_rows)
        if r < RE - 1:
            nwt = p['enc_msg_W'][r + 1].T
            nb = _row(p['enc_msg_b'][r + 1])
        else:
            nwt, nb = None, None
        h, m = _upd(aggp[:N], aggp[NP:NP + N], h,
                    p['enc_upd_W'][r].T, _row(p['enc_upd_b'][r]),
                    enc_g, nwt, nb)

    mu_n, lv_n, sum_h, kl_sum = _mu_head(
        h, p['mu_W'].T, _row(p['mu_b']), p['lv_W'].T, _row(p['lv_b']))

    gr = sum_h / N
    mu_g = (jnp.maximum(gr @ p['gmu_W1'].T + p['gmu_b1'], 0.0)
            @ p['gmu_W2'].T + p['gmu_b2'])[0]
    lv_g = (jnp.maximum(gr @ p['glv_W1'].T + p['glv_b1'], 0.0)
            @ p['glv_W2'].T + p['glv_b2'])[0]
    z_g = mu_g + eps_g * jnp.exp(0.5 * lv_g)
    kl_node = -0.5 * kl_sum[0, 0] / (N * Z)
    kl_graph = -0.5 * jnp.mean(1.0 + lv_g - mu_g ** 2 - jnp.exp(lv_g))

    proj_wt = p['proj_W'].T
    beff = _row(p['proj_b']) + z_g[None, :] @ proj_wt[Z:]

    h, m = _z_proj(mu_n, lv_n, eps_n, proj_wt[:Z], beff,
                   p['dec_msg_W'][0].T, _row(p['dec_msg_b'][0]))
    for r in range(RD):
        aggp = agg_sc(m, src3d, dst3d, zeros_rows)
        if r < RD - 1:
            nwt = p['dec_msg_W'][r + 1].T
            nb = _row(p['dec_msg_b'][r + 1])
        else:
            nwt, nb = None, None
        h, m = _upd(aggp[:N], aggp[NP:NP + N], h,
                    p['dec_upd_W'][r].T, _row(p['dec_upd_b'][r]),
                    dec_g, nwt, nb)
    hd = h

    gidx = jnp.concatenate([src, neg[0], dst, neg[1]])
    gv = gather_sc(hd, gidx)

    w1 = p['emlp_W1']
    inv_tau = 1.0 / p['tau']
    w2col = (p['emlp_W2'] * inv_tau).T
    b2s = (p['emlp_b2'] * inv_tau + p['logit_bias']).reshape(1, 1)
    pos_sum, neg_sum = _edge_score(
        gv, w1[:, 0:S].T, w1[:, S:2 * S].T, w1[:, 2 * S:3 * S].T,
        w1[:, 3 * S:].T, _row(p['emlp_b1']), w2col, b2s)

    pw = 12.0
    recon = -(pw * pos_sum[0, 0] + neg_sum[0, 0]) / (2.0 * E)
    kl = kl_node + kl_graph
    loss = recon + kl
    return loss, recon, kl


# speedup vs baseline: 6.1760x; 1.3747x over previous
import functools

import jax
import jax.numpy as jnp
from jax import lax
from jax.experimental import pallas as pl
from jax.experimental.pallas import tpu as pltpu
from jax.experimental.pallas import tpu_sc as plsc

S = 64
IN = 128
Z = 32
G = 16
RE = 6
RD = 2
N = 10000
E = 320000

ROW_BLK = 1000
N_ROW_BLKS = N // ROW_BLK

NC = 2
NS = 16
NW = NC * NS

EPW = E // NW
CH = 80
NCH = EPW // CH
NP = 10240
NPS = NP // NS

E2 = 2 * E
GPW = E2 // NW
GCH = 80
GNCH = GPW // GCH

EDGE_BLK = 4000
N_EDGE_BLKS = (2 * E) // EDGE_BLK
N_POS_BLKS = E // EDGE_BLK


def _dot(a, b):
    return jnp.dot(a, b, preferred_element_type=jnp.float32)


def _full_spec(shape):
    return pl.BlockSpec(shape, lambda i: (0,) * len(shape))


def _row_spec(last):
    return pl.BlockSpec((ROW_BLK, last), lambda i: (i, 0))


def _in_proj_kernel(x_ref, w_ref, b_ref, mw_ref, mb_ref, h_ref, m_ref):
    h = jnp.maximum(_dot(x_ref[...], w_ref[...]) + b_ref[...], 0.0)
    h_ref[...] = h
    m_ref[...] = jnp.maximum(_dot(h, mw_ref[...]) + mb_ref[...], 0.0)


def _in_proj(x, wt, brow, mwt, mbrow):
    return pl.pallas_call(
        _in_proj_kernel,
        grid=(N_ROW_BLKS,),
        in_specs=[
            _row_spec(IN),
            _full_spec((IN, S)),
            _full_spec((1, S)),
            _full_spec((S, S)),
            _full_spec((1, S)),
        ],
        out_specs=[_row_spec(S), _row_spec(S)],
        out_shape=[
            jax.ShapeDtypeStruct((N, S), jnp.float32),
            jax.ShapeDtypeStruct((N, S), jnp.float32),
        ],
    )(x, wt, brow, mwt, mbrow)


def _upd_kernel(fuse_next, a0_ref, a1_ref, h_ref, uw_ref, ub_ref,
                wr_ref, wz_ref, wn_ref, ur_ref, uz_ref, un_ref,
                brz_ref, bzz_ref, bin_ref, bhn_ref, nw_ref, nb_ref,
                *out_refs):
    agg = a0_ref[...] + a1_ref[...]
    msg = jnp.maximum(_dot(agg, uw_ref[...]) + ub_ref[...], 0.0)
    h = h_ref[...]
    r = jax.nn.sigmoid(_dot(msg, wr_ref[...]) + _dot(h, ur_ref[...]) + brz_ref[...])
    z = jax.nn.sigmoid(_dot(msg, wz_ref[...]) + _dot(h, uz_ref[...]) + bzz_ref[...])
    n = jnp.tanh(_dot(msg, wn_ref[...]) + bin_ref[...]
                 + r * (_dot(h, un_ref[...]) + bhn_ref[...]))
    hn = (1.0 - z) * n + z * h
    out_refs[0][...] = hn
    if fuse_next:
        out_refs[1][...] = jnp.maximum(_dot(hn, nw_ref[...]) + nb_ref[...], 0.0)


def _upd(a0, a1, h, uwt, ubrow, gw, nwt, nbrow):
    fuse_next = nwt is not None
    if not fuse_next:
        nwt = jnp.zeros((S, S), jnp.float32)
        nbrow = jnp.zeros((1, S), jnp.float32)
    n_out = 2 if fuse_next else 1
    outs = pl.pallas_call(
        functools.partial(_upd_kernel, fuse_next),
        grid=(N_ROW_BLKS,),
        in_specs=[
            _row_spec(S),
            _row_spec(S),
            _row_spec(S),
            _full_spec((S, S)), _full_spec((1, S)),
            _full_spec((S, S)), _full_spec((S, S)), _full_spec((S, S)),
            _full_spec((S, S)), _full_spec((S, S)), _full_spec((S, S)),
            _full_spec((1, S)), _full_spec((1, S)), _full_spec((1, S)),
            _full_spec((1, S)),
            _full_spec((S, S)), _full_spec((1, S)),
        ],
        out_specs=[_row_spec(S)] * n_out,
        out_shape=[jax.ShapeDtypeStruct((N, S), jnp.float32)] * n_out,
    )(a0, a1, h, uwt, ubrow, *gw, nwt, nbrow)
    if fuse_next:
        return outs[0], outs[1]
    return outs[0], None


def _mu_kernel(h_ref, mw_ref, mb_ref, lw_ref, lb_ref,
               mu_ref, lv_ref, sum_ref, kl_ref):
    h = h_ref[...]
    mu = _dot(h, mw_ref[...]) + mb_ref[...]
    lv = _dot(h, lw_ref[...]) + lb_ref[...]
    mu_ref[...] = mu
    lv_ref[...] = lv

    @pl.when(pl.program_id(0) == 0)
    def _():
        sum_ref[...] = jnp.zeros_like(sum_ref)
        kl_ref[...] = jnp.zeros_like(kl_ref)

    sum_ref[...] += jnp.sum(h, axis=0, keepdims=True)
    kl_ref[...] += jnp.sum(1.0 + lv - mu * mu - jnp.exp(lv)).reshape(1, 1)


def _mu_head(h, mwt, mbrow, lwt, lbrow):
    return pl.pallas_call(
        _mu_kernel,
        grid=(N_ROW_BLKS,),
        in_specs=[
            _row_spec(S),
            _full_spec((S, Z)), _full_spec((1, Z)),
            _full_spec((S, Z)), _full_spec((1, Z)),
        ],
        out_specs=[
            _row_spec(Z), _row_spec(Z),
            _full_spec((1, S)), _full_spec((1, 1)),
        ],
        out_shape=[
            jax.ShapeDtypeStruct((N, Z), jnp.float32),
            jax.ShapeDtypeStruct((N, Z), jnp.float32),
            jax.ShapeDtypeStruct((1, S), jnp.float32),
            jax.ShapeDtypeStruct((1, 1), jnp.float32),
        ],
    )(h, mwt, mbrow, lwt, lbrow)


def _z_kernel(mu_ref, lv_ref, eps_ref, pw_ref, be_ref, mw_ref, mb_ref,
              h_ref, m_ref):
    zn = mu_ref[...] + eps_ref[...] * jnp.exp(0.5 * lv_ref[...])
    h = jnp.maximum(_dot(zn, pw_ref[...]) + be_ref[...], 0.0)
    h_ref[...] = h
    m_ref[...] = jnp.maximum(_dot(h, mw_ref[...]) + mb_ref[...], 0.0)


def _z_proj(mu_n, lv_n, eps_n, pwt_n, beff, mwt, mbrow):
    return pl.pallas_call(
        _z_kernel,
        grid=(N_ROW_BLKS,),
        in_specs=[
            _row_spec(Z), _row_spec(Z), _row_spec(Z),
            _full_spec((Z, S)), _full_spec((1, S)),
            _full_spec((S, S)), _full_spec((1, S)),
        ],
        out_specs=[_row_spec(S), _row_spec(S)],
        out_shape=[
            jax.ShapeDtypeStruct((N, S), jnp.float32),
            jax.ShapeDtypeStruct((N, S), jnp.float32),
        ],
    )(mu_n, lv_n, eps_n, pwt_n, beff, mwt, mbrow)


def _edge_kernel(gv_ref, wab_ref, wc_ref, wd_ref,
                 b1_ref, w2_ref, b2_ref, pos_ref, neg_ref):
    blk = gv_ref[...]
    hu = blk[:, :S]
    hv = blk[:, S:]
    t = (_dot(blk, wab_ref[...])
         + _dot(jnp.abs(hu - hv), wc_ref[...]) + _dot(hu * hv, wd_ref[...])
         + b1_ref[...])
    t = jnp.maximum(t, 0.0)
    logits = _dot(t, w2_ref[...]) + b2_ref[...]

    @pl.when(pl.program_id(0) == 0)
    def _():
        pos_ref[...] = jnp.zeros_like(pos_ref)
        neg_ref[...] = jnp.zeros_like(neg_ref)

    is_pos = pl.program_id(0) < N_POS_BLKS
    pos_term = jnp.sum(jax.nn.log_sigmoid(logits)).reshape(1, 1)
    neg_term = jnp.sum(jax.nn.log_sigmoid(-logits)).reshape(1, 1)
    pos_ref[...] += jnp.where(is_pos, pos_term, jnp.zeros((1, 1), jnp.float32))
    neg_ref[...] += jnp.where(is_pos, jnp.zeros((1, 1), jnp.float32), neg_term)


def _edge_score(gv, wab, wc, wd, b1row, w2col, b2s):
    return pl.pallas_call(
        _edge_kernel,
        grid=(N_EDGE_BLKS,),
        in_specs=[
            pl.BlockSpec((EDGE_BLK, 2 * S), lambda i: (i, 0)),
            _full_spec((2 * S, S)),
            _full_spec((S, S)), _full_spec((S, S)),
            _full_spec((1, S)), _full_spec((S, 1)), _full_spec((1, 1)),
        ],
        out_specs=[_full_spec((1, 1)), _full_spec((1, 1))],
        out_shape=[
            jax.ShapeDtypeStruct((1, 1), jnp.float32),
            jax.ShapeDtypeStruct((1, 1), jnp.float32),
        ],
    )(gv, wab, wc, wd, b1row, w2col, b2s)


@functools.lru_cache(maxsize=None)
def _build_agg_sc():
    mesh = plsc.VectorSubcoreMesh(core_axis_name="c", subcore_axis_name="s")

    @functools.partial(
        pl.kernel,
        mesh=mesh,
        compiler_params=pltpu.CompilerParams(use_tc_tiling_on_sc=False),
        out_type=jax.ShapeDtypeStruct((NC * NP, S), jnp.float32),
        scratch_types=[
            pltpu.VMEM((NCH, CH), jnp.int32),
            pltpu.VMEM((NCH, CH), jnp.int32),
            pltpu.VMEM((CH, S), jnp.float32),
            pltpu.VMEM((CH, S), jnp.float32),
            pltpu.VMEM_SHARED((NP, S), jnp.float32),
            pltpu.SemaphoreType.DMA,
            pltpu.SemaphoreType.DMA,
        ],
    )
    def agg_sc(m_hbm, src_hbm, dst_hbm, zeros_hbm, out_hbm,
               idx_s, idx_d, rows_a, rows_b, agg_sh, gs_a, gs_b):
        cid = lax.axis_index("c")
        sid = lax.axis_index("s")
        wid = sid * NC + cid
        pltpu.sync_copy(zeros_hbm, agg_sh.at[pl.ds(sid * NPS, NPS)])
        pltpu.sync_copy(src_hbm.at[wid], idx_s)
        pltpu.sync_copy(dst_hbm.at[wid], idx_d)
        plsc.subcore_barrier()

        def start_g(j, buf, sem):
            pltpu.async_copy(m_hbm.at[idx_s.at[j]], buf, sem)

        def wait_g(buf, sem):
            pltpu.make_async_copy(m_hbm.at[pl.ds(0, CH)], buf, sem).wait()

        start_g(0, rows_a, gs_a)

        def body(i, carry):
            j = 2 * i
            wait_g(rows_a, gs_a)
            start_g(j + 1, rows_b, gs_b)
            pltpu.sync_copy(rows_a, agg_sh.at[idx_d.at[j]], add=True)
            wait_g(rows_b, gs_b)
            start_g(j + 2, rows_a, gs_a)
            pltpu.sync_copy(rows_b, agg_sh.at[idx_d.at[j + 1]], add=True)
            return carry

        lax.fori_loop(0, (NCH - 1) // 2, body, 0)
        wait_g(rows_a, gs_a)
        pltpu.sync_copy(rows_a, agg_sh.at[idx_d.at[NCH - 1]], add=True)
        plsc.subcore_barrier()
        pltpu.sync_copy(agg_sh.at[pl.ds(sid * NPS, NPS)],
                        out_hbm.at[pl.ds(cid * NP + sid * NPS, NPS)])

    return agg_sc


@functools.lru_cache(maxsize=None)
def _build_gather_sc():
    mesh = plsc.VectorSubcoreMesh(core_axis_name="c", subcore_axis_name="s")

    @functools.partial(
        pl.kernel,
        mesh=mesh,
        compiler_params=pltpu.CompilerParams(use_tc_tiling_on_sc=False),
        out_type=jax.ShapeDtypeStruct((E2, 2 * S), jnp.float32),
        scratch_types=[
            pltpu.VMEM((GNCH, GCH), jnp.int32),
            pltpu.VMEM((GNCH, GCH), jnp.int32),
            pltpu.VMEM((GCH, S), jnp.float32),
            pltpu.VMEM((GCH, S), jnp.float32),
            pltpu.VMEM((GCH, S), jnp.float32),
            pltpu.VMEM((GCH, S), jnp.float32),
            pltpu.SemaphoreType.DMA,
            pltpu.SemaphoreType.DMA,
        ],
    )
    def gather_sc(hd_hbm, idxu_hbm, idxv_hbm, out_hbm,
                  idx_u, idx_v, ua, va, ub, vb, gs_a, gs_b):
        cid = lax.axis_index("c")
        sid = lax.axis_index("s")
        wid = sid * NC + cid
        pltpu.sync_copy(idxu_hbm.at[wid], idx_u)
        pltpu.sync_copy(idxv_hbm.at[wid], idx_v)

        def start_g(j, bu, bv, sem):
            pltpu.async_copy(hd_hbm.at[idx_u.at[j]], bu, sem)
            pltpu.async_copy(hd_hbm.at[idx_v.at[j]], bv, sem)

        def wait_g(bu, bv, sem):
            pltpu.make_async_copy(hd_hbm.at[pl.ds(0, GCH)], bu, sem).wait()
            pltpu.make_async_copy(hd_hbm.at[pl.ds(0, GCH)], bv, sem).wait()

        def store(j, bu, bv):
            base = wid * GPW + j * GCH
            pltpu.sync_copy(bu, out_hbm.at[pl.ds(base, GCH), pl.ds(0, S)])
            pltpu.sync_copy(bv, out_hbm.at[pl.ds(base, GCH), pl.ds(S, S)])

        start_g(0, ua, va, gs_a)

        def body(i, carry):
            j = 2 * i
            wait_g(ua, va, gs_a)
            start_g(j + 1, ub, vb, gs_b)
            store(j, ua, va)
            wait_g(ub, vb, gs_b)

            @pl.when(i < (GNCH // 2) - 1)
            def _():
                start_g(j + 2, ua, va, gs_a)

            store(j + 1, ub, vb)
            return carry

        lax.fori_loop(0, GNCH // 2, body, 0)

    return gather_sc


def _row(v):
    return v[None, :]


def kernel(x, edge_index, params, neg_ratio):
    p = params
    f32 = jnp.float32

    eps_n = jax.random.normal(jax.random.key(11), (N, Z), dtype=f32)
    eps_g = jax.random.normal(jax.random.key(12), (G,), dtype=f32)
    neg = jax.random.randint(jax.random.key(13), (2, E), 0, N,
                             dtype=edge_index.dtype)
    neg = neg % (neg_ratio * N)

    src = edge_index[0]
    dst = edge_index[1]
    src3d = src.reshape(NW, NCH, CH)
    dst3d = dst.reshape(NW, NCH, CH)
    zeros_rows = jnp.zeros((NPS, S), f32)

    def grup(prefix):
        wih = p[prefix + '_Wih']; whh = p[prefix + '_Whh']
        bih = p[prefix + '_bih']; bhh = p[prefix + '_bhh']
        return (
            wih[0:S].T, wih[S:2 * S].T, wih[2 * S:].T,
            whh[0:S].T, whh[S:2 * S].T, whh[2 * S:].T,
            _row(bih[0:S] + bhh[0:S]), _row(bih[S:2 * S] + bhh[S:2 * S]),
            _row(bih[2 * S:]), _row(bhh[2 * S:]),
        )

    enc_g = grup('enc_gru')
    dec_g = grup('dec_gru')

    agg_sc = _build_agg_sc()
    gather_sc = _build_gather_sc()

    h, m = _in_proj(x, p['enc_in_W'].T, _row(p['enc_in_b']),
                    p['enc_msg_W'][0].T, _row(p['enc_msg_b'][0]))
    for r in range(RE):
        aggp = agg_sc(m, src3d, dst3d, zeros_rows)
        if r < RE - 1:
            nwt = p['enc_msg_W'][r + 1].T
            nb = _row(p['enc_msg_b'][r + 1])
        else:
            nwt, nb = None, None
        h, m = _upd(aggp[:N], aggp[NP:NP + N], h,
                    p['enc_upd_W'][r].T, _row(p['enc_upd_b'][r]),
                    enc_g, nwt, nb)

    mu_n, lv_n, sum_h, kl_sum = _mu_head(
        h, p['mu_W'].T, _row(p['mu_b']), p['lv_W'].T, _row(p['lv_b']))

    gr = sum_h / N
    mu_g = (jnp.maximum(gr @ p['gmu_W1'].T + p['gmu_b1'], 0.0)
            @ p['gmu_W2'].T + p['gmu_b2'])[0]
    lv_g = (jnp.maximum(gr @ p['glv_W1'].T + p['glv_b1'], 0.0)
            @ p['glv_W2'].T + p['glv_b2'])[0]
    z_g = mu_g + eps_g * jnp.exp(0.5 * lv_g)
    kl_node = -0.5 * kl_sum[0, 0] / (N * Z)
    kl_graph = -0.5 * jnp.mean(1.0 + lv_g - mu_g ** 2 - jnp.exp(lv_g))

    proj_wt = p['proj_W'].T
    beff = _row(p['proj_b']) + z_g[None, :] @ proj_wt[Z:]

    h, m = _z_proj(mu_n, lv_n, eps_n, proj_wt[:Z], beff,
                   p['dec_msg_W'][0].T, _row(p['dec_msg_b'][0]))
    for r in range(RD):
        aggp = agg_sc(m, src3d, dst3d, zeros_rows)
        if r < RD - 1:
            nwt = p['dec_msg_W'][r + 1].T
            nb = _row(p['dec_msg_b'][r + 1])
        else:
            nwt, nb = None, None
        h, m = _upd(aggp[:N], aggp[NP:NP + N], h,
                    p['dec_upd_W'][r].T, _row(p['dec_upd_b'][r]),
                    dec_g, nwt, nb)
    hd = h

    idxu3d = jnp.concatenate([src, neg[0]]).reshape(NW, GNCH, GCH)
    idxv3d = jnp.concatenate([dst, neg[1]]).reshape(NW, GNCH, GCH)
    gv = gather_sc(hd, idxu3d, idxv3d)

    w1 = p['emlp_W1']
    inv_tau = 1.0 / p['tau']
    w2col = (p['emlp_W2'] * inv_tau).T
    b2s = (p['emlp_b2'] * inv_tau + p['logit_bias']).reshape(1, 1)
    wab = jnp.concatenate([w1[:, 0:S].T, w1[:, S:2 * S].T], axis=0)
    pos_sum, neg_sum = _edge_score(
        gv, wab, w1[:, 2 * S:3 * S].T,
        w1[:, 3 * S:].T, _row(p['emlp_b1']), w2col, b2s)

    pw = 12.0
    recon = -(pw * pos_sum[0, 0] + neg_sum[0, 0]) / (2.0 * E)
    kl = kl_node + kl_graph
    loss = recon + kl
    return loss, recon, kl
